# baseline (speedup 1.0000x reference)
import jax
import jax.numpy as jnp
from jax import lax
from jax.experimental import pallas as pl
from jax.experimental.pallas import tpu as pltpu
from jax.experimental.pallas import tpu_sc as plsc

BATCH = 16384
DIM = 16
_NC = 2
_NS = 16
_NW = _NC * _NS
_RPW = BATCH // _NW
_STAGE = 128
_NSTAGE = _RPW // _STAGE
_BPAD = 100096


def _mirt_body(stu_ref, exer_ref, thg_ref, alg_ref, beg_ref, out_ref,
               sidx, eidx, tgi, agi, bgi, th, al, be, ov, sem):
    wid = lax.axis_index("s") * _NC + lax.axis_index("c")
    base = wid * _RPW
    pltpu.sync_copy(stu_ref.at[pl.ds(base, _RPW)], sidx)
    pltpu.sync_copy(exer_ref.at[pl.ds(base, _RPW)], eidx)

    lane = lax.iota(jnp.int32, 16)

    for s in range(_NSTAGE):
        def idx_body(k, carry):
            o = pl.multiple_of(s * _STAGE + k * 16, 16)
            ko = pl.multiple_of(k * 16, 16)
            sv = sidx[pl.ds(o, 16)]
            ev = eidx[pl.ds(o, 16)]
            tgi[pl.ds(ko, 16)] = lax.shift_right_logical(sv, 3)
            agi[pl.ds(ko, 16)] = lax.shift_right_logical(ev, 3)
            bgi[pl.ds(ko, 16)] = lax.shift_right_logical(ev, 7)
            return carry

        lax.fori_loop(0, _STAGE // 16, idx_body, 0)

        c_th = pltpu.async_copy(thg_ref.at[tgi], th, sem)
        c_al = pltpu.async_copy(alg_ref.at[agi], al, sem)
        c_be = pltpu.async_copy(beg_ref.at[bgi], be, sem)
        c_th.wait()
        c_al.wait()
        c_be.wait()

        def chunk_body(k, carry):
            o = pl.multiple_of(s * _STAGE + k * 16, 16)
            ko = pl.multiple_of(k * 16, 16)
            sv = sidx[pl.ds(o, 16)]
            ev = eidx[pl.ds(o, 16)]
            rows = k * 16 + lane
            tcb = lax.shift_left(jnp.bitwise_and(sv, 7), 4)
            acb = lax.shift_left(jnp.bitwise_and(ev, 7), 4)
            bcol = jnp.bitwise_and(ev, 127)
            acc = jnp.zeros((16,), jnp.float32)
            for c in range(DIM):
                t = plsc.load_gather(th, [rows, tcb + c])
                a = plsc.load_gather(al, [rows, acb + c])
                acc = acc + t * a
            b = plsc.load_gather(be, [rows, bcol])
            x = acc - b
            ov[pl.ds(o, 16)] = 1.0 / (1.0 + jnp.exp(-x))
            return carry

        lax.fori_loop(0, _STAGE // 16, chunk_body, 0)

    pltpu.sync_copy(ov, out_ref.at[pl.ds(base, _RPW)])


def kernel(stu_id, exer_id, kn_emb, thetas, alphas, betas):
    del kn_emb
    thg = thetas.reshape(-1, 128)
    alg = alphas.reshape(-1, 128)
    beg = jnp.pad(betas.reshape(-1), (0, _BPAD - betas.shape[0])).reshape(
        -1, 128)
    mesh = plsc.VectorSubcoreMesh(core_axis_name="c", subcore_axis_name="s",
                                  num_cores=_NC, num_subcores=_NS)
    return pl.kernel(
        _mirt_body,
        out_type=jax.ShapeDtypeStruct((BATCH,), jnp.float32),
        mesh=mesh,
        compiler_params=pltpu.CompilerParams(needs_layout_passes=False),
        scratch_types=[
            pltpu.VMEM((_RPW,), jnp.int32),
            pltpu.VMEM((_RPW,), jnp.int32),
            pltpu.VMEM((_STAGE,), jnp.int32),
            pltpu.VMEM((_STAGE,), jnp.int32),
            pltpu.VMEM((_STAGE,), jnp.int32),
            pltpu.VMEM((_STAGE, 128), jnp.float32),
            pltpu.VMEM((_STAGE, 128), jnp.float32),
            pltpu.VMEM((_STAGE, 128), jnp.float32),
            pltpu.VMEM((_RPW,), jnp.float32),
            pltpu.SemaphoreType.DMA,
        ],
    )(stu_id, exer_id, thg, alg, beg)


# speedup vs baseline: 1.5746x; 1.5746x over previous
import jax
import jax.numpy as jnp
from jax import lax
from jax.experimental import pallas as pl
from jax.experimental.pallas import tpu as pltpu
from jax.experimental.pallas import tpu_sc as plsc

BATCH = 16384
DIM = 16
_NC = 2
_NS = 16
_NW = _NC * _NS
_RPW = BATCH // _NW
_CH = 128
_NCHUNK = _RPW // _CH


def _mirt_body(stu_ref, exer_ref, thetas_ref, alphas_ref, betas_ref, out_ref,
               sidx, eidx, th, al, be, ov, sem_t, sem_a, sem_b):
    wid = lax.axis_index("s") * _NC + lax.axis_index("c")
    base = wid * _RPW
    pltpu.sync_copy(stu_ref.at[pl.ds(base, _RPW)], sidx)
    pltpu.sync_copy(exer_ref.at[pl.ds(base, _RPW)], eidx)

    c_be = pltpu.async_copy(betas_ref.at[eidx], be, sem_b)

    lane = lax.iota(jnp.int32, 16)

    for s in range(_NCHUNK):
        def fire(k, carry):
            o = pl.multiple_of(s * _CH + k * 16, 16)
            sv = sidx[pl.ds(o, 16)]
            ev = eidx[pl.ds(o, 16)]
            for l in range(16):
                dst = pl.ds(k * 16 + l, 1)
                pltpu.async_copy(thetas_ref.at[pl.ds(sv[l], 1), :],
                                 th.at[dst, :], sem_t)
                pltpu.async_copy(alphas_ref.at[pl.ds(ev[l], 1), :],
                                 al.at[dst, :], sem_a)
            return carry

        lax.fori_loop(0, _CH // 16, fire, 0)
        pltpu.make_async_copy(thetas_ref.at[pl.ds(0, _CH), :], th,
                              sem_t).wait()
        pltpu.make_async_copy(alphas_ref.at[pl.ds(0, _CH), :], al,
                              sem_a).wait()
        if s == 0:
            c_be.wait()

        def chunk_body(k, carry):
            o = pl.multiple_of(s * _CH + k * 16, 16)
            rows = k * 16 + lane
            acc = jnp.zeros((16,), jnp.float32)
            for c in range(DIM):
                cv = jnp.full((16,), c, jnp.int32)
                acc = acc + plsc.load_gather(th, [rows, cv]) * \
                    plsc.load_gather(al, [rows, cv])
            x = acc - be[pl.ds(o, 16)]
            ov[pl.ds(o, 16)] = 1.0 / (1.0 + jnp.exp(-x))
            return carry

        lax.fori_loop(0, _CH // 16, chunk_body, 0)

    pltpu.sync_copy(ov, out_ref.at[pl.ds(base, _RPW)])


def kernel(stu_id, exer_id, kn_emb, thetas, alphas, betas):
    del kn_emb
    mesh = plsc.VectorSubcoreMesh(core_axis_name="c", subcore_axis_name="s",
                                  num_cores=_NC, num_subcores=_NS)
    return pl.kernel(
        _mirt_body,
        out_type=jax.ShapeDtypeStruct((BATCH,), jnp.float32),
        mesh=mesh,
        compiler_params=pltpu.CompilerParams(needs_layout_passes=False),
        scratch_types=[
            pltpu.VMEM((_RPW,), jnp.int32),
            pltpu.VMEM((_RPW,), jnp.int32),
            pltpu.VMEM((_CH, DIM), jnp.float32),
            pltpu.VMEM((_CH, DIM), jnp.float32),
            pltpu.VMEM((_RPW,), jnp.float32),
            pltpu.VMEM((_RPW,), jnp.float32),
            pltpu.SemaphoreType.DMA,
            pltpu.SemaphoreType.DMA,
            pltpu.SemaphoreType.DMA,
        ],
    )(stu_id, exer_id, thetas, alphas, betas.reshape(-1))
